# baseline (speedup 1.0000x reference)
import functools

import jax
import jax.numpy as jnp
import numpy as np
from jax import lax
from jax.experimental import pallas as pl
from jax.experimental.pallas import tpu as pltpu
from jax.experimental.pallas import tpu_sc as plsc

_B, _T, _C = 64, 2048, 128


def _static_crop_params(B, T):
    rng = np.random.RandomState(0)
    crop_l = int(rng.randint(2, T + 1))
    crop_left = int(rng.randint(T - crop_l + 1))
    crop_right = crop_left + crop_l
    crop_eleft = int(rng.randint(crop_left + 1))
    crop_eright = int(rng.randint(crop_right, T + 1))
    crop_offset = rng.randint(-crop_eleft, T - crop_eright + 1, size=B)
    return crop_l, crop_left, crop_right, crop_eleft, crop_eright, crop_offset


(_CROP_L, _CROP_LEFT, _CROP_RIGHT, _CROP_ELEFT, _CROP_ERIGHT,
 _CROP_OFFSET) = _static_crop_params(_B, _T)
_W1 = _CROP_RIGHT - _CROP_ELEFT
_W2 = _CROP_ERIGHT - _CROP_LEFT
_S1 = [int(v) for v in (_CROP_OFFSET + _CROP_ELEFT)]
_S2 = [int(v) for v in (_CROP_OFFSET + _CROP_LEFT)]

_D21 = _CROP_LEFT - _CROP_ELEFT
_WU = _D21 + _W2
assert 0 <= _D21 <= _W1 <= _WU

_NUM_CORES = 2
_NUM_SUBCORES = 16
_NW = _NUM_CORES * _NUM_SUBCORES
_ROWS_PER_W = _B // _NW

_NCH = 6
_CHR = -(-_WU // _NCH)
_NBUF = 3


def _row_plan(b):
    s1 = _S1[b]
    chunks = []
    for a in range(0, _WU, _CHR):
        rows = min(_CHR, _WU - a)
        pieces = []
        if a < _W1:
            pieces.append((0, 0, a, min(rows, _W1 - a)))
        if a + rows > _D21:
            lo = max(a, _D21)
            pieces.append((1, lo - a, lo - _D21, a + rows - lo))
        chunks.append((s1 + a, rows, pieces))
    return chunks


@functools.partial(
    pl.kernel,
    out_type=(
        jax.ShapeDtypeStruct((_W1, _B, _C), jnp.float32),
        jax.ShapeDtypeStruct((_W2, _B, _C), jnp.float32),
    ),
    mesh=plsc.VectorSubcoreMesh(
        core_axis_name="c", subcore_axis_name="s",
        num_cores=_NUM_CORES, num_subcores=_NUM_SUBCORES),
    scratch_types=[
        pltpu.VMEM((_NBUF, _CHR, _C), jnp.float32),
        pltpu.SemaphoreType.DMA,
        pltpu.SemaphoreType.DMA,
    ],
    compiler_params=pltpu.CompilerParams(use_tc_tiling_on_sc=False),
)
def _crop_sc(x_hbm, out1_hbm, out2_hbm, bufs, in_sem, out_sem):
    wid = lax.axis_index("s") * _NUM_CORES + lax.axis_index("c")
    for w in range(_NW):
        @pl.when(wid == w)
        def _copies(w=w):
            out_refs = (out1_hbm, out2_hbm)
            cin, cout = [], []
            for j in range(_ROWS_PER_W):
                b = w * _ROWS_PER_W + j
                for src_row, rows, pieces in _row_plan(b):
                    buf = bufs.at[len(cin) % _NBUF]
                    cin.append(pltpu.make_async_copy(
                        x_hbm.at[b, pl.ds(src_row, rows)],
                        buf.at[pl.ds(0, rows)], in_sem))
                    cout.append([pltpu.make_async_copy(
                        buf.at[pl.ds(vrow, prow)],
                        out_refs[oi].at[pl.ds(dst, prow), b], out_sem)
                        for oi, vrow, dst, prow in pieces])
            n = len(cin)
            for i in range(min(_NBUF - 1, n)):
                cin[i].start()
            for i in range(n):
                if i + _NBUF - 1 < n:
                    if i > 0:
                        for c in cout[i - 1]:
                            c.wait()
                    cin[i + _NBUF - 1].start()
                cin[i].wait()
                for c in cout[i]:
                    c.start()
            for i in range(max(0, n - _NBUF), n):
                for c in cout[i]:
                    c.wait()


def kernel(x):
    t1, t2 = _crop_sc(x)
    return (jnp.transpose(t1, (1, 0, 2)), jnp.transpose(t2, (1, 0, 2)))


# speedup vs baseline: 1.0153x; 1.0153x over previous
import functools

import jax
import jax.numpy as jnp
import numpy as np
from jax import lax
from jax.experimental import pallas as pl
from jax.experimental.pallas import tpu as pltpu
from jax.experimental.pallas import tpu_sc as plsc

_B, _T, _C = 64, 2048, 128


def _static_crop_params(B, T):
    rng = np.random.RandomState(0)
    crop_l = int(rng.randint(2, T + 1))
    crop_left = int(rng.randint(T - crop_l + 1))
    crop_right = crop_left + crop_l
    crop_eleft = int(rng.randint(crop_left + 1))
    crop_eright = int(rng.randint(crop_right, T + 1))
    crop_offset = rng.randint(-crop_eleft, T - crop_eright + 1, size=B)
    return crop_l, crop_left, crop_right, crop_eleft, crop_eright, crop_offset


(_CROP_L, _CROP_LEFT, _CROP_RIGHT, _CROP_ELEFT, _CROP_ERIGHT,
 _CROP_OFFSET) = _static_crop_params(_B, _T)
_W1 = _CROP_RIGHT - _CROP_ELEFT
_W2 = _CROP_ERIGHT - _CROP_LEFT
_S1 = [int(v) for v in (_CROP_OFFSET + _CROP_ELEFT)]
_S2 = [int(v) for v in (_CROP_OFFSET + _CROP_LEFT)]

_D21 = _CROP_LEFT - _CROP_ELEFT
_WU = _D21 + _W2
assert 0 <= _D21 <= _W1 <= _WU

_NUM_CORES = 2
_NUM_SUBCORES = 16
_NW = _NUM_CORES * _NUM_SUBCORES
_ROWS_PER_W = _B // _NW

_NCH = 4
_CHR = -(-_WU // _NCH)


def _row_plan(b):
    s1 = _S1[b]
    chunks = []
    for a in range(0, _WU, _CHR):
        rows = min(_CHR, _WU - a)
        pieces = []
        if a < _W1:
            pieces.append((0, 0, a, min(rows, _W1 - a)))
        if a + rows > _D21:
            lo = max(a, _D21)
            pieces.append((1, lo - a, lo - _D21, a + rows - lo))
        chunks.append((s1 + a, rows, pieces))
    return chunks


@functools.partial(
    pl.kernel,
    out_type=(
        jax.ShapeDtypeStruct((_W1, _B, _C), jnp.float32),
        jax.ShapeDtypeStruct((_W2, _B, _C), jnp.float32),
    ),
    mesh=plsc.VectorSubcoreMesh(
        core_axis_name="c", subcore_axis_name="s",
        num_cores=_NUM_CORES, num_subcores=_NUM_SUBCORES),
    scratch_types=[
        pltpu.VMEM((2, _CHR, _C), jnp.float32),
        pltpu.SemaphoreType.DMA,
        pltpu.SemaphoreType.DMA,
    ],
    compiler_params=pltpu.CompilerParams(use_tc_tiling_on_sc=False),
)
def _crop_sc(x_hbm, out1_hbm, out2_hbm, bufs, in_sem, out_sem):
    wid = lax.axis_index("s") * _NUM_CORES + lax.axis_index("c")
    for w in range(_NW):
        @pl.when(wid == w)
        def _copies(w=w):
            out_refs = (out1_hbm, out2_hbm)
            cin, cout = [], []
            for j in range(_ROWS_PER_W):
                b = w * _ROWS_PER_W + j
                for src_row, rows, pieces in _row_plan(b):
                    buf = bufs.at[len(cin) % 2]
                    cin.append(pltpu.make_async_copy(
                        x_hbm.at[b, pl.ds(src_row, rows)],
                        buf.at[pl.ds(0, rows)], in_sem))
                    cout.append([pltpu.make_async_copy(
                        buf.at[pl.ds(vrow, prow)],
                        out_refs[oi].at[pl.ds(dst, prow), b], out_sem)
                        for oi, vrow, dst, prow in pieces])
            n = len(cin)
            cin[0].start()
            for i in range(n):
                if i > 0:
                    for c in cout[i - 1]:
                        c.wait()
                if i + 1 < n:
                    cin[i + 1].start()
                cin[i].wait()
                for c in cout[i]:
                    c.start()
            for c in cout[n - 1]:
                c.wait()


def kernel(x):
    t1, t2 = _crop_sc(x)
    return (jnp.transpose(t1, (1, 0, 2)), jnp.transpose(t2, (1, 0, 2)))
